# Initial kernel scaffold; baseline (speedup 1.0000x reference)
import jax
import jax.numpy as jnp
from jax.experimental import pallas as pl


def kernel(points_2d, camera_indices, grouping_indices, point_indices, camera_pps, depths_ref, intrs, points_3d, ref_poses, rel_poses):
    raise NotImplementedError("write your pallas kernel here")


# speedup vs baseline: 3.9886x; 3.9886x over previous
import functools

import jax
import jax.numpy as jnp
from jax import lax
from jax.experimental import pallas as pl
from jax.experimental.pallas import tpu as pltpu
from jax.experimental.pallas import tpu_sc as plsc

T = 2000
CHUNK = 80
NCH = T // CHUNK
VPT = T // 16
NW = 32
DEPTH_W = 0.1
EPS = 1e-6


def _c16(v):
    return jnp.full((16,), v, jnp.int32)


def _make_sc_call(n):
    nt = n // T

    mesh = plsc.VectorSubcoreMesh(core_axis_name="c", subcore_axis_name="s",
                                  num_cores=2, num_subcores=16)

    @functools.partial(
        pl.kernel,
        out_type=jax.ShapeDtypeStruct((3 * n,), jnp.float32),
        mesh=mesh,
        compiler_params=pltpu.CompilerParams(use_tc_tiling_on_sc=False,
                                             needs_layout_passes=False),
        scratch_types=[
            pltpu.VMEM((T,), jnp.int32),
            pltpu.VMEM((T,), jnp.int32),
            pltpu.VMEM((T,), jnp.int32),
            pltpu.VMEM((T,), jnp.int32),
            pltpu.VMEM((T,), jnp.float32),
            pltpu.VMEM((2 * T,), jnp.float32),
            pltpu.VMEM((T, 8), jnp.float32),
            pltpu.VMEM((T, 8), jnp.float32),
            pltpu.VMEM((3 * T,), jnp.float32),
            pltpu.VMEM((8, 7), jnp.float32),
            pltpu.VMEM((8, 2), jnp.float32),
            pltpu.VMEM((8, 2), jnp.float32),
            pltpu.SemaphoreType.DMA,
            pltpu.SemaphoreType.DMA,
        ],
    )
    def sc_call(p2d_hbm, g0_hbm, mem_hbm, cam_hbm, pti_hbm, dep_hbm,
                refp_hbm, ptsp_hbm, rel_hbm, intr_hbm, pp_hbm, out_hbm,
                g0_v, pti_v, mem_v, cam_v, dep_v, p2d_v,
                ref_rows, pt_rows, out_v, rel_v, intr_v, pp_v,
                sem_in, sem_g):
        wid = lax.axis_index("s") * 2 + lax.axis_index("c")

        pltpu.sync_copy(rel_hbm, rel_v)
        pltpu.sync_copy(intr_hbm, intr_v)
        pltpu.sync_copy(pp_hbm, pp_v)

        n_tiles_w = (nt + NW - 1 - wid) // NW

        def tile_body(k, carry):
            tile = wid + k * NW
            tbase = tile * T

            cps = [
                pltpu.async_copy(g0_hbm.at[pl.ds(tbase, T)], g0_v, sem_in),
                pltpu.async_copy(pti_hbm.at[pl.ds(tbase, T)], pti_v, sem_in),
                pltpu.async_copy(mem_hbm.at[pl.ds(tbase, T)], mem_v, sem_in),
                pltpu.async_copy(cam_hbm.at[pl.ds(tbase, T)], cam_v, sem_in),
                pltpu.async_copy(dep_hbm.at[pl.ds(tbase, T)], dep_v, sem_in),
                pltpu.async_copy(p2d_hbm.at[pl.ds(2 * tbase, 2 * T)], p2d_v, sem_in),
            ]
            for cp in cps:
                cp.wait()

            gcps = []
            for j in range(NCH):
                gcps.append(pltpu.async_copy(
                    refp_hbm.at[g0_v.at[pl.ds(j * CHUNK, CHUNK)]],
                    ref_rows.at[pl.ds(j * CHUNK, CHUNK), :], sem_g))
                gcps.append(pltpu.async_copy(
                    ptsp_hbm.at[pti_v.at[pl.ds(j * CHUNK, CHUNK)]],
                    pt_rows.at[pl.ds(j * CHUNK, CHUNK), :], sem_g))
            for cp in gcps:
                cp.wait()

            def vec_body(i, c2):
                row = i * 16 + lax.iota(jnp.int32, 16)
                mem16 = plsc.load_gather(mem_v, [row])
                cam16 = plsc.load_gather(cam_v, [row])
                dep16 = plsc.load_gather(dep_v, [row])
                u2 = plsc.load_gather(p2d_v, [2 * row])
                v2 = plsc.load_gather(p2d_v, [2 * row + 1])

                rtx = plsc.load_gather(rel_v, [mem16, _c16(0)])
                rty = plsc.load_gather(rel_v, [mem16, _c16(1)])
                rtz = plsc.load_gather(rel_v, [mem16, _c16(2)])
                rqx = plsc.load_gather(rel_v, [mem16, _c16(3)])
                rqy = plsc.load_gather(rel_v, [mem16, _c16(4)])
                rqz = plsc.load_gather(rel_v, [mem16, _c16(5)])
                rqw = plsc.load_gather(rel_v, [mem16, _c16(6)])

                ttx = plsc.load_gather(ref_rows, [row, _c16(0)])
                tty = plsc.load_gather(ref_rows, [row, _c16(1)])
                ttz = plsc.load_gather(ref_rows, [row, _c16(2)])
                tqx = plsc.load_gather(ref_rows, [row, _c16(3)])
                tqy = plsc.load_gather(ref_rows, [row, _c16(4)])
                tqz = plsc.load_gather(ref_rows, [row, _c16(5)])
                tqw = plsc.load_gather(ref_rows, [row, _c16(6)])

                qw = rqw * tqw - rqx * tqx - rqy * tqy - rqz * tqz
                qx = rqw * tqx + rqx * tqw + rqy * tqz - rqz * tqy
                qy = rqw * tqy + rqy * tqw + rqz * tqx - rqx * tqz
                qz = rqw * tqz + rqz * tqw + rqx * tqy - rqy * tqx

                ax = 2.0 * (rqy * ttz - rqz * tty)
                ay = 2.0 * (rqz * ttx - rqx * ttz)
                az = 2.0 * (rqx * tty - rqy * ttx)
                tx = ttx + rqw * ax + (rqy * az - rqz * ay) + rtx
                ty = tty + rqw * ay + (rqz * ax - rqx * az) + rty
                tz = ttz + rqw * az + (rqx * ay - rqy * ax) + rtz

                px = plsc.load_gather(pt_rows, [row, _c16(0)])
                py = plsc.load_gather(pt_rows, [row, _c16(1)])
                pz = plsc.load_gather(pt_rows, [row, _c16(2)])

                bx = 2.0 * (qy * pz - qz * py)
                by = 2.0 * (qz * px - qx * pz)
                bz = 2.0 * (qx * py - qy * px)
                cxx = px + qw * bx + (qy * bz - qz * by) + tx
                cyy = py + qw * by + (qz * bx - qx * bz) + ty
                czz = pz + qw * bz + (qx * by - qy * bx) + tz

                fx = plsc.load_gather(intr_v, [cam16, _c16(0)])
                fy = plsc.load_gather(intr_v, [cam16, _c16(1)])
                cpx = plsc.load_gather(pp_v, [cam16, _c16(0)])
                cpy = plsc.load_gather(pp_v, [cam16, _c16(1)])

                lu = cxx / czz * fx + cpx - u2
                lv = cyy / czz * fy + cpy - v2
                ld = (1.0 / (czz + EPS) - dep16) * DEPTH_W

                plsc.store_scatter(out_v, [3 * row], lu)
                plsc.store_scatter(out_v, [3 * row + 1], lv)
                plsc.store_scatter(out_v, [3 * row + 2], ld)
                return c2

            lax.fori_loop(0, VPT, vec_body, 0)

            pltpu.sync_copy(out_v, out_hbm.at[pl.ds(3 * tbase, 3 * T)])
            return carry

        lax.fori_loop(0, n_tiles_w, tile_body, 0)

    return sc_call


def kernel(points_2d, camera_indices, grouping_indices, point_indices,
           camera_pps, depths_ref, intrs, points_3d, ref_poses, rel_poses):
    n = points_2d.shape[0]
    p2d = points_2d.astype(jnp.float32).reshape(-1)
    g0 = grouping_indices[:, 0].astype(jnp.int32)
    mem = grouping_indices[:, 1].astype(jnp.int32)
    cam = camera_indices.astype(jnp.int32)
    pti = point_indices.astype(jnp.int32)
    dep = depths_ref.astype(jnp.float32)
    refp = jnp.pad(ref_poses.astype(jnp.float32), ((0, 0), (0, 1)))
    ptsp = jnp.pad(points_3d.astype(jnp.float32), ((0, 0), (0, 5)))

    outf = _make_sc_call(n)(
        p2d, g0, mem, cam, pti, dep, refp, ptsp,
        rel_poses.astype(jnp.float32), intrs.astype(jnp.float32),
        camera_pps.astype(jnp.float32))
    return outf.reshape(n, 3)


# speedup vs baseline: 33.2537x; 8.3371x over previous
import functools

import jax
import jax.numpy as jnp
from jax import lax
from jax.experimental import pallas as pl
from jax.experimental.pallas import tpu as pltpu
from jax.experimental.pallas import tpu_sc as plsc

T = 2000
CHUNK = 80
NCH = T // CHUNK
VPT = T // 16
NW = 32
DEPTH_W = 0.1
EPS = 1e-6

_MESH = plsc.VectorSubcoreMesh(core_axis_name="c", subcore_axis_name="s",
                               num_cores=2, num_subcores=16)
_CPARAMS = pltpu.CompilerParams(use_tc_tiling_on_sc=False,
                                needs_layout_passes=False)


def _c16(v):
    return jnp.full((16,), v, jnp.int32)


def _make_pack_points(npts):
    nt = npts // T

    @functools.partial(
        pl.kernel,
        out_type=jax.ShapeDtypeStruct((npts * 8,), jnp.float32),
        mesh=_MESH,
        compiler_params=_CPARAMS,
        scratch_types=[
            pltpu.VMEM((T,), jnp.float32),
            pltpu.VMEM((T,), jnp.float32),
            pltpu.VMEM((T,), jnp.float32),
            pltpu.VMEM((8 * T,), jnp.float32),
            pltpu.SemaphoreType.DMA,
        ],
    )
    def pack_call(x_hbm, y_hbm, z_hbm, out_hbm, x_v, y_v, z_v, row_v, sem):
        wid = lax.axis_index("s") * 2 + lax.axis_index("c")
        n_tiles_w = (nt + NW - 1 - wid) // NW

        def tile_body(k, carry):
            tbase = (wid + k * NW) * T
            cps = [
                pltpu.async_copy(x_hbm.at[pl.ds(tbase, T)], x_v, sem),
                pltpu.async_copy(y_hbm.at[pl.ds(tbase, T)], y_v, sem),
                pltpu.async_copy(z_hbm.at[pl.ds(tbase, T)], z_v, sem),
            ]
            for cp in cps:
                cp.wait()

            def vec_body(i, c2):
                row = i * 16 + lax.iota(jnp.int32, 16)
                plsc.store_scatter(row_v, [8 * row], plsc.load_gather(x_v, [row]))
                plsc.store_scatter(row_v, [8 * row + 1], plsc.load_gather(y_v, [row]))
                plsc.store_scatter(row_v, [8 * row + 2], plsc.load_gather(z_v, [row]))
                return c2

            lax.fori_loop(0, VPT, vec_body, 0)
            pltpu.sync_copy(row_v, out_hbm.at[pl.ds(8 * tbase, 8 * T)])
            return carry

        lax.fori_loop(0, n_tiles_w, tile_body, 0)

    return pack_call


def _make_sc_call(n):
    nt = n // T

    @functools.partial(
        pl.kernel,
        out_type=[jax.ShapeDtypeStruct((n,), jnp.float32)] * 3,
        mesh=_MESH,
        compiler_params=_CPARAMS,
        scratch_types=[
            pltpu.VMEM((T,), jnp.int32),
            pltpu.VMEM((T,), jnp.int32),
            pltpu.VMEM((T,), jnp.int32),
            pltpu.VMEM((T,), jnp.int32),
            pltpu.VMEM((T,), jnp.float32),
            pltpu.VMEM((T,), jnp.float32),
            pltpu.VMEM((T,), jnp.float32),
            pltpu.VMEM((T, 8), jnp.float32),
            pltpu.VMEM((T, 8), jnp.float32),
            pltpu.VMEM((T,), jnp.float32),
            pltpu.VMEM((T,), jnp.float32),
            pltpu.VMEM((T,), jnp.float32),
            pltpu.VMEM((8, 7), jnp.float32),
            pltpu.VMEM((8, 2), jnp.float32),
            pltpu.VMEM((8, 2), jnp.float32),
            pltpu.SemaphoreType.DMA,
            pltpu.SemaphoreType.DMA,
        ],
    )
    def sc_call(u2d_hbm, v2d_hbm, g0_hbm, mem_hbm, cam_hbm, pti_hbm, dep_hbm,
                refp_hbm, ptsp_hbm, rel_hbm, intr_hbm, pp_hbm,
                outu_hbm, outv_hbm, outd_hbm,
                g0_v, pti_v, mem_v, cam_v, dep_v, u2d_v, v2d_v,
                ref_rows, pt_rows, outu_v, outv_v, outd_v, rel_v, intr_v, pp_v,
                sem_in, sem_g):
        wid = lax.axis_index("s") * 2 + lax.axis_index("c")

        pltpu.sync_copy(rel_hbm, rel_v)
        pltpu.sync_copy(intr_hbm, intr_v)
        pltpu.sync_copy(pp_hbm, pp_v)

        n_tiles_w = (nt + NW - 1 - wid) // NW

        def tile_body(k, carry):
            tile = wid + k * NW
            tbase = tile * T

            cps = [
                pltpu.async_copy(g0_hbm.at[pl.ds(tbase, T)], g0_v, sem_in),
                pltpu.async_copy(pti_hbm.at[pl.ds(tbase, T)], pti_v, sem_in),
                pltpu.async_copy(mem_hbm.at[pl.ds(tbase, T)], mem_v, sem_in),
                pltpu.async_copy(cam_hbm.at[pl.ds(tbase, T)], cam_v, sem_in),
                pltpu.async_copy(dep_hbm.at[pl.ds(tbase, T)], dep_v, sem_in),
                pltpu.async_copy(u2d_hbm.at[pl.ds(tbase, T)], u2d_v, sem_in),
                pltpu.async_copy(v2d_hbm.at[pl.ds(tbase, T)], v2d_v, sem_in),
            ]
            for cp in cps:
                cp.wait()

            gcps = []
            for j in range(NCH):
                gcps.append(pltpu.async_copy(
                    refp_hbm.at[g0_v.at[pl.ds(j * CHUNK, CHUNK)]],
                    ref_rows.at[pl.ds(j * CHUNK, CHUNK), :], sem_g))
                gcps.append(pltpu.async_copy(
                    ptsp_hbm.at[pti_v.at[pl.ds(j * CHUNK, CHUNK)]],
                    pt_rows.at[pl.ds(j * CHUNK, CHUNK), :], sem_g))
            for cp in gcps:
                cp.wait()

            def vec_body(i, c2):
                row = i * 16 + lax.iota(jnp.int32, 16)
                mem16 = plsc.load_gather(mem_v, [row])
                cam16 = plsc.load_gather(cam_v, [row])
                dep16 = plsc.load_gather(dep_v, [row])
                u2 = plsc.load_gather(u2d_v, [row])
                v2 = plsc.load_gather(v2d_v, [row])

                rtx = plsc.load_gather(rel_v, [mem16, _c16(0)])
                rty = plsc.load_gather(rel_v, [mem16, _c16(1)])
                rtz = plsc.load_gather(rel_v, [mem16, _c16(2)])
                rqx = plsc.load_gather(rel_v, [mem16, _c16(3)])
                rqy = plsc.load_gather(rel_v, [mem16, _c16(4)])
                rqz = plsc.load_gather(rel_v, [mem16, _c16(5)])
                rqw = plsc.load_gather(rel_v, [mem16, _c16(6)])

                ttx = plsc.load_gather(ref_rows, [row, _c16(0)])
                tty = plsc.load_gather(ref_rows, [row, _c16(1)])
                ttz = plsc.load_gather(ref_rows, [row, _c16(2)])
                tqx = plsc.load_gather(ref_rows, [row, _c16(3)])
                tqy = plsc.load_gather(ref_rows, [row, _c16(4)])
                tqz = plsc.load_gather(ref_rows, [row, _c16(5)])
                tqw = plsc.load_gather(ref_rows, [row, _c16(6)])

                qw = rqw * tqw - rqx * tqx - rqy * tqy - rqz * tqz
                qx = rqw * tqx + rqx * tqw + rqy * tqz - rqz * tqy
                qy = rqw * tqy + rqy * tqw + rqz * tqx - rqx * tqz
                qz = rqw * tqz + rqz * tqw + rqx * tqy - rqy * tqx

                ax = 2.0 * (rqy * ttz - rqz * tty)
                ay = 2.0 * (rqz * ttx - rqx * ttz)
                az = 2.0 * (rqx * tty - rqy * ttx)
                tx = ttx + rqw * ax + (rqy * az - rqz * ay) + rtx
                ty = tty + rqw * ay + (rqz * ax - rqx * az) + rty
                tz = ttz + rqw * az + (rqx * ay - rqy * ax) + rtz

                px = plsc.load_gather(pt_rows, [row, _c16(0)])
                py = plsc.load_gather(pt_rows, [row, _c16(1)])
                pz = plsc.load_gather(pt_rows, [row, _c16(2)])

                bx = 2.0 * (qy * pz - qz * py)
                by = 2.0 * (qz * px - qx * pz)
                bz = 2.0 * (qx * py - qy * px)
                cxx = px + qw * bx + (qy * bz - qz * by) + tx
                cyy = py + qw * by + (qz * bx - qx * bz) + ty
                czz = pz + qw * bz + (qx * by - qy * bx) + tz

                fx = plsc.load_gather(intr_v, [cam16, _c16(0)])
                fy = plsc.load_gather(intr_v, [cam16, _c16(1)])
                cpx = plsc.load_gather(pp_v, [cam16, _c16(0)])
                cpy = plsc.load_gather(pp_v, [cam16, _c16(1)])

                lu = cxx / czz * fx + cpx - u2
                lv = cyy / czz * fy + cpy - v2
                ld = (1.0 / (czz + EPS) - dep16) * DEPTH_W

                plsc.store_scatter(outu_v, [row], lu)
                plsc.store_scatter(outv_v, [row], lv)
                plsc.store_scatter(outd_v, [row], ld)
                return c2

            lax.fori_loop(0, VPT, vec_body, 0)

            pltpu.sync_copy(outu_v, outu_hbm.at[pl.ds(tbase, T)])
            pltpu.sync_copy(outv_v, outv_hbm.at[pl.ds(tbase, T)])
            pltpu.sync_copy(outd_v, outd_hbm.at[pl.ds(tbase, T)])
            return carry

        lax.fori_loop(0, n_tiles_w, tile_body, 0)

    return sc_call


def kernel(points_2d, camera_indices, grouping_indices, point_indices,
           camera_pps, depths_ref, intrs, points_3d, ref_poses, rel_poses):
    n = points_2d.shape[0]
    npts = points_3d.shape[0]
    u2d = points_2d[:, 0].astype(jnp.float32)
    v2d = points_2d[:, 1].astype(jnp.float32)
    g0 = grouping_indices[:, 0].astype(jnp.int32)
    mem = grouping_indices[:, 1].astype(jnp.int32)
    cam = camera_indices.astype(jnp.int32)
    pti = point_indices.astype(jnp.int32)
    dep = depths_ref.astype(jnp.float32)
    refp = jnp.pad(ref_poses.astype(jnp.float32), ((0, 0), (0, 1)))

    p3 = points_3d.astype(jnp.float32)
    ptsp = _make_pack_points(npts)(p3[:, 0], p3[:, 1], p3[:, 2]).reshape(npts, 8)

    lu, lv, ld = _make_sc_call(n)(
        u2d, v2d, g0, mem, cam, pti, dep, refp, ptsp,
        rel_poses.astype(jnp.float32), intrs.astype(jnp.float32),
        camera_pps.astype(jnp.float32))
    return jnp.stack([lu, lv, ld], axis=-1)


# speedup vs baseline: 33.3576x; 1.0031x over previous
import functools

import jax
import jax.numpy as jnp
from jax import lax
from jax.experimental import pallas as pl
from jax.experimental.pallas import tpu as pltpu
from jax.experimental.pallas import tpu_sc as plsc

T = 2000
CHUNK = 2000
NCH = T // CHUNK
VPT = T // 16
NW = 32
DEPTH_W = 0.1
EPS = 1e-6

_MESH = plsc.VectorSubcoreMesh(core_axis_name="c", subcore_axis_name="s",
                               num_cores=2, num_subcores=16)
_CPARAMS = pltpu.CompilerParams(use_tc_tiling_on_sc=False,
                                needs_layout_passes=False)


def _c16(v):
    return jnp.full((16,), v, jnp.int32)


def _make_pack_points(npts):
    nt = npts // T

    @functools.partial(
        pl.kernel,
        out_type=jax.ShapeDtypeStruct((npts * 8,), jnp.float32),
        mesh=_MESH,
        compiler_params=_CPARAMS,
        scratch_types=[
            pltpu.VMEM((T,), jnp.float32),
            pltpu.VMEM((T,), jnp.float32),
            pltpu.VMEM((T,), jnp.float32),
            pltpu.VMEM((8 * T,), jnp.float32),
            pltpu.SemaphoreType.DMA,
        ],
    )
    def pack_call(x_hbm, y_hbm, z_hbm, out_hbm, x_v, y_v, z_v, row_v, sem):
        wid = lax.axis_index("s") * 2 + lax.axis_index("c")
        n_tiles_w = (nt + NW - 1 - wid) // NW

        def tile_body(k, carry):
            tbase = (wid + k * NW) * T
            cps = [
                pltpu.async_copy(x_hbm.at[pl.ds(tbase, T)], x_v, sem),
                pltpu.async_copy(y_hbm.at[pl.ds(tbase, T)], y_v, sem),
                pltpu.async_copy(z_hbm.at[pl.ds(tbase, T)], z_v, sem),
            ]
            for cp in cps:
                cp.wait()

            def vec_body(i, c2):
                row = i * 16 + lax.iota(jnp.int32, 16)
                plsc.store_scatter(row_v, [8 * row], plsc.load_gather(x_v, [row]))
                plsc.store_scatter(row_v, [8 * row + 1], plsc.load_gather(y_v, [row]))
                plsc.store_scatter(row_v, [8 * row + 2], plsc.load_gather(z_v, [row]))
                return c2

            lax.fori_loop(0, VPT, vec_body, 0)
            pltpu.sync_copy(row_v, out_hbm.at[pl.ds(8 * tbase, 8 * T)])
            return carry

        lax.fori_loop(0, n_tiles_w, tile_body, 0)

    return pack_call


def _make_sc_call(n):
    nt = n // T

    @functools.partial(
        pl.kernel,
        out_type=[jax.ShapeDtypeStruct((n,), jnp.float32)] * 3,
        mesh=_MESH,
        compiler_params=_CPARAMS,
        scratch_types=[
            pltpu.VMEM((T,), jnp.int32),
            pltpu.VMEM((T,), jnp.int32),
            pltpu.VMEM((T,), jnp.int32),
            pltpu.VMEM((T,), jnp.int32),
            pltpu.VMEM((T,), jnp.float32),
            pltpu.VMEM((T,), jnp.float32),
            pltpu.VMEM((T,), jnp.float32),
            pltpu.VMEM((T, 8), jnp.float32),
            pltpu.VMEM((T, 8), jnp.float32),
            pltpu.VMEM((T,), jnp.float32),
            pltpu.VMEM((T,), jnp.float32),
            pltpu.VMEM((T,), jnp.float32),
            pltpu.VMEM((8, 7), jnp.float32),
            pltpu.VMEM((8, 2), jnp.float32),
            pltpu.VMEM((8, 2), jnp.float32),
            pltpu.SemaphoreType.DMA,
            pltpu.SemaphoreType.DMA,
        ],
    )
    def sc_call(u2d_hbm, v2d_hbm, g0_hbm, mem_hbm, cam_hbm, pti_hbm, dep_hbm,
                refp_hbm, ptsp_hbm, rel_hbm, intr_hbm, pp_hbm,
                outu_hbm, outv_hbm, outd_hbm,
                g0_v, pti_v, mem_v, cam_v, dep_v, u2d_v, v2d_v,
                ref_rows, pt_rows, outu_v, outv_v, outd_v, rel_v, intr_v, pp_v,
                sem_in, sem_g):
        wid = lax.axis_index("s") * 2 + lax.axis_index("c")

        pltpu.sync_copy(rel_hbm, rel_v)
        pltpu.sync_copy(intr_hbm, intr_v)
        pltpu.sync_copy(pp_hbm, pp_v)

        n_tiles_w = (nt + NW - 1 - wid) // NW

        def tile_body(k, carry):
            tile = wid + k * NW
            tbase = tile * T

            cps = [
                pltpu.async_copy(g0_hbm.at[pl.ds(tbase, T)], g0_v, sem_in),
                pltpu.async_copy(pti_hbm.at[pl.ds(tbase, T)], pti_v, sem_in),
                pltpu.async_copy(mem_hbm.at[pl.ds(tbase, T)], mem_v, sem_in),
                pltpu.async_copy(cam_hbm.at[pl.ds(tbase, T)], cam_v, sem_in),
                pltpu.async_copy(dep_hbm.at[pl.ds(tbase, T)], dep_v, sem_in),
                pltpu.async_copy(u2d_hbm.at[pl.ds(tbase, T)], u2d_v, sem_in),
                pltpu.async_copy(v2d_hbm.at[pl.ds(tbase, T)], v2d_v, sem_in),
            ]
            for cp in cps:
                cp.wait()

            gcps = []
            for j in range(NCH):
                gcps.append(pltpu.async_copy(
                    refp_hbm.at[g0_v.at[pl.ds(j * CHUNK, CHUNK)]],
                    ref_rows.at[pl.ds(j * CHUNK, CHUNK), :], sem_g))
                gcps.append(pltpu.async_copy(
                    ptsp_hbm.at[pti_v.at[pl.ds(j * CHUNK, CHUNK)]],
                    pt_rows.at[pl.ds(j * CHUNK, CHUNK), :], sem_g))
            for cp in gcps:
                cp.wait()

            def vec_body(i, c2):
                row = i * 16 + lax.iota(jnp.int32, 16)
                mem16 = plsc.load_gather(mem_v, [row])
                cam16 = plsc.load_gather(cam_v, [row])
                dep16 = plsc.load_gather(dep_v, [row])
                u2 = plsc.load_gather(u2d_v, [row])
                v2 = plsc.load_gather(v2d_v, [row])

                rtx = plsc.load_gather(rel_v, [mem16, _c16(0)])
                rty = plsc.load_gather(rel_v, [mem16, _c16(1)])
                rtz = plsc.load_gather(rel_v, [mem16, _c16(2)])
                rqx = plsc.load_gather(rel_v, [mem16, _c16(3)])
                rqy = plsc.load_gather(rel_v, [mem16, _c16(4)])
                rqz = plsc.load_gather(rel_v, [mem16, _c16(5)])
                rqw = plsc.load_gather(rel_v, [mem16, _c16(6)])

                ttx = plsc.load_gather(ref_rows, [row, _c16(0)])
                tty = plsc.load_gather(ref_rows, [row, _c16(1)])
                ttz = plsc.load_gather(ref_rows, [row, _c16(2)])
                tqx = plsc.load_gather(ref_rows, [row, _c16(3)])
                tqy = plsc.load_gather(ref_rows, [row, _c16(4)])
                tqz = plsc.load_gather(ref_rows, [row, _c16(5)])
                tqw = plsc.load_gather(ref_rows, [row, _c16(6)])

                qw = rqw * tqw - rqx * tqx - rqy * tqy - rqz * tqz
                qx = rqw * tqx + rqx * tqw + rqy * tqz - rqz * tqy
                qy = rqw * tqy + rqy * tqw + rqz * tqx - rqx * tqz
                qz = rqw * tqz + rqz * tqw + rqx * tqy - rqy * tqx

                ax = 2.0 * (rqy * ttz - rqz * tty)
                ay = 2.0 * (rqz * ttx - rqx * ttz)
                az = 2.0 * (rqx * tty - rqy * ttx)
                tx = ttx + rqw * ax + (rqy * az - rqz * ay) + rtx
                ty = tty + rqw * ay + (rqz * ax - rqx * az) + rty
                tz = ttz + rqw * az + (rqx * ay - rqy * ax) + rtz

                px = plsc.load_gather(pt_rows, [row, _c16(0)])
                py = plsc.load_gather(pt_rows, [row, _c16(1)])
                pz = plsc.load_gather(pt_rows, [row, _c16(2)])

                bx = 2.0 * (qy * pz - qz * py)
                by = 2.0 * (qz * px - qx * pz)
                bz = 2.0 * (qx * py - qy * px)
                cxx = px + qw * bx + (qy * bz - qz * by) + tx
                cyy = py + qw * by + (qz * bx - qx * bz) + ty
                czz = pz + qw * bz + (qx * by - qy * bx) + tz

                fx = plsc.load_gather(intr_v, [cam16, _c16(0)])
                fy = plsc.load_gather(intr_v, [cam16, _c16(1)])
                cpx = plsc.load_gather(pp_v, [cam16, _c16(0)])
                cpy = plsc.load_gather(pp_v, [cam16, _c16(1)])

                lu = cxx / czz * fx + cpx - u2
                lv = cyy / czz * fy + cpy - v2
                ld = (1.0 / (czz + EPS) - dep16) * DEPTH_W

                plsc.store_scatter(outu_v, [row], lu)
                plsc.store_scatter(outv_v, [row], lv)
                plsc.store_scatter(outd_v, [row], ld)
                return c2

            lax.fori_loop(0, VPT, vec_body, 0)

            ocps = [
                pltpu.async_copy(outu_v, outu_hbm.at[pl.ds(tbase, T)], sem_in),
                pltpu.async_copy(outv_v, outv_hbm.at[pl.ds(tbase, T)], sem_in),
                pltpu.async_copy(outd_v, outd_hbm.at[pl.ds(tbase, T)], sem_in),
            ]
            for cp in ocps:
                cp.wait()
            return carry

        lax.fori_loop(0, n_tiles_w, tile_body, 0)

    return sc_call


def kernel(points_2d, camera_indices, grouping_indices, point_indices,
           camera_pps, depths_ref, intrs, points_3d, ref_poses, rel_poses):
    n = points_2d.shape[0]
    npts = points_3d.shape[0]
    u2d = points_2d[:, 0].astype(jnp.float32)
    v2d = points_2d[:, 1].astype(jnp.float32)
    g0 = grouping_indices[:, 0].astype(jnp.int32)
    mem = grouping_indices[:, 1].astype(jnp.int32)
    cam = camera_indices.astype(jnp.int32)
    pti = point_indices.astype(jnp.int32)
    dep = depths_ref.astype(jnp.float32)
    refp = jnp.pad(ref_poses.astype(jnp.float32), ((0, 0), (0, 1)))

    p3 = points_3d.astype(jnp.float32)
    ptsp = _make_pack_points(npts)(p3[:, 0], p3[:, 1], p3[:, 2]).reshape(npts, 8)

    lu, lv, ld = _make_sc_call(n)(
        u2d, v2d, g0, mem, cam, pti, dep, refp, ptsp,
        rel_poses.astype(jnp.float32), intrs.astype(jnp.float32),
        camera_pps.astype(jnp.float32))
    return jnp.stack([lu, lv, ld], axis=-1)


# speedup vs baseline: 46.3969x; 1.3909x over previous
import functools

import jax
import jax.numpy as jnp
from jax import lax
from jax.experimental import pallas as pl
from jax.experimental.pallas import tpu as pltpu
from jax.experimental.pallas import tpu_sc as plsc

T = 1600
VPT = T // 16
NW = 32
TP = 2000
TC_ = 1600
NG = 20000
NM = 8
DEPTH_W = 0.1
EPS = 1e-6

_MESH = plsc.VectorSubcoreMesh(core_axis_name="c", subcore_axis_name="s",
                               num_cores=2, num_subcores=16)
_CPARAMS = pltpu.CompilerParams(use_tc_tiling_on_sc=False,
                                needs_layout_passes=False)


def _c16(v):
    return jnp.full((16,), v, jnp.int32)


def _make_prep(npts):
    npt_tiles = npts // TP
    ncmp = NG * NM
    ncmp_tiles = ncmp // TC_
    gpt = TC_ // NM

    @functools.partial(
        pl.kernel,
        out_type=[jax.ShapeDtypeStruct((npts * 8,), jnp.float32),
                  jax.ShapeDtypeStruct((ncmp * 8,), jnp.float32)],
        mesh=_MESH,
        compiler_params=_CPARAMS,
        scratch_types=[
            pltpu.VMEM((TP,), jnp.float32),
            pltpu.VMEM((TP,), jnp.float32),
            pltpu.VMEM((TP,), jnp.float32),
            pltpu.VMEM((8 * TP,), jnp.float32),
            pltpu.VMEM((gpt, 8), jnp.float32),
            pltpu.VMEM((8, 7), jnp.float32),
            pltpu.VMEM((8 * TC_,), jnp.float32),
            pltpu.SemaphoreType.DMA,
        ],
    )
    def prep_call(x_hbm, y_hbm, z_hbm, refp_hbm, rel_hbm, out_hbm, comp_hbm,
                  x_v, y_v, z_v, row_v, ref_v, rel_v, comp_v, sem):
        wid = lax.axis_index("s") * 2 + lax.axis_index("c")
        pltpu.sync_copy(rel_hbm, rel_v)

        n_tiles_w = (npt_tiles + NW - 1 - wid) // NW

        def tile_body(k, carry):
            tbase = (wid + k * NW) * TP
            cps = [
                pltpu.async_copy(x_hbm.at[pl.ds(tbase, TP)], x_v, sem),
                pltpu.async_copy(y_hbm.at[pl.ds(tbase, TP)], y_v, sem),
                pltpu.async_copy(z_hbm.at[pl.ds(tbase, TP)], z_v, sem),
            ]
            for cp in cps:
                cp.wait()

            def vec_body(i, c2):
                row = i * 16 + lax.iota(jnp.int32, 16)
                plsc.store_scatter(row_v, [8 * row], plsc.load_gather(x_v, [row]))
                plsc.store_scatter(row_v, [8 * row + 1], plsc.load_gather(y_v, [row]))
                plsc.store_scatter(row_v, [8 * row + 2], plsc.load_gather(z_v, [row]))
                return c2

            lax.fori_loop(0, TP // 16, vec_body, 0)
            pltpu.sync_copy(row_v, out_hbm.at[pl.ds(8 * tbase, 8 * TP)])
            return carry

        lax.fori_loop(0, n_tiles_w, tile_body, 0)

        n_ctiles_w = (ncmp_tiles + NW - 1 - wid) // NW

        def ctile_body(k, carry):
            t = wid + k * NW
            pltpu.sync_copy(refp_hbm.at[pl.ds(gpt * t, gpt), :], ref_v)

            def cvec_body(i, c2):
                lrow = i * 16 + lax.iota(jnp.int32, 16)
                gl = lax.shift_right_logical(lrow, 3)
                m = lax.bitwise_and(lrow, 7)

                ttx = plsc.load_gather(ref_v, [gl, _c16(0)])
                tty = plsc.load_gather(ref_v, [gl, _c16(1)])
                ttz = plsc.load_gather(ref_v, [gl, _c16(2)])
                tqx = plsc.load_gather(ref_v, [gl, _c16(3)])
                tqy = plsc.load_gather(ref_v, [gl, _c16(4)])
                tqz = plsc.load_gather(ref_v, [gl, _c16(5)])
                tqw = plsc.load_gather(ref_v, [gl, _c16(6)])

                rtx = plsc.load_gather(rel_v, [m, _c16(0)])
                rty = plsc.load_gather(rel_v, [m, _c16(1)])
                rtz = plsc.load_gather(rel_v, [m, _c16(2)])
                rqx = plsc.load_gather(rel_v, [m, _c16(3)])
                rqy = plsc.load_gather(rel_v, [m, _c16(4)])
                rqz = plsc.load_gather(rel_v, [m, _c16(5)])
                rqw = plsc.load_gather(rel_v, [m, _c16(6)])

                qw = rqw * tqw - rqx * tqx - rqy * tqy - rqz * tqz
                qx = rqw * tqx + rqx * tqw + rqy * tqz - rqz * tqy
                qy = rqw * tqy + rqy * tqw + rqz * tqx - rqx * tqz
                qz = rqw * tqz + rqz * tqw + rqx * tqy - rqy * tqx

                ax = 2.0 * (rqy * ttz - rqz * tty)
                ay = 2.0 * (rqz * ttx - rqx * ttz)
                az = 2.0 * (rqx * tty - rqy * ttx)
                tx = ttx + rqw * ax + (rqy * az - rqz * ay) + rtx
                ty = tty + rqw * ay + (rqz * ax - rqx * az) + rty
                tz = ttz + rqw * az + (rqx * ay - rqy * ax) + rtz

                plsc.store_scatter(comp_v, [8 * lrow], tx)
                plsc.store_scatter(comp_v, [8 * lrow + 1], ty)
                plsc.store_scatter(comp_v, [8 * lrow + 2], tz)
                plsc.store_scatter(comp_v, [8 * lrow + 3], qx)
                plsc.store_scatter(comp_v, [8 * lrow + 4], qy)
                plsc.store_scatter(comp_v, [8 * lrow + 5], qz)
                plsc.store_scatter(comp_v, [8 * lrow + 6], qw)
                return c2

            lax.fori_loop(0, TC_ // 16, cvec_body, 0)
            pltpu.sync_copy(comp_v, comp_hbm.at[pl.ds(8 * TC_ * t, 8 * TC_)])
            return carry

        lax.fori_loop(0, n_ctiles_w, ctile_body, 0)

    return prep_call


def _make_sc_call(n):
    nt = n // T
    base_slots = nt // NW
    rem = nt - base_slots * NW
    nslots = base_slots + 1

    @functools.partial(
        pl.kernel,
        out_type=[jax.ShapeDtypeStruct((n,), jnp.float32)] * 3,
        mesh=_MESH,
        compiler_params=_CPARAMS,
        scratch_types=[
            [pltpu.VMEM((T,), jnp.int32)] * 2,
            [pltpu.VMEM((T,), jnp.int32)] * 2,
            [pltpu.VMEM((T,), jnp.int32)] * 2,
            [pltpu.VMEM((T,), jnp.float32)] * 2,
            [pltpu.VMEM((T,), jnp.float32)] * 2,
            [pltpu.VMEM((T,), jnp.float32)] * 2,
            [pltpu.VMEM((T, 8), jnp.float32)] * 2,
            [pltpu.VMEM((T, 8), jnp.float32)] * 2,
            [pltpu.VMEM((T,), jnp.float32)] * 2,
            [pltpu.VMEM((T,), jnp.float32)] * 2,
            [pltpu.VMEM((T,), jnp.float32)] * 2,
            pltpu.VMEM((8, 2), jnp.float32),
            pltpu.VMEM((8, 2), jnp.float32),
            pltpu.SemaphoreType.DMA,
            [pltpu.SemaphoreType.DMA] * 2,
            [pltpu.SemaphoreType.DMA] * 2,
            [pltpu.SemaphoreType.DMA] * 2,
        ],
    )
    def sc_call(u2d_hbm, v2d_hbm, cidx_hbm, cam_hbm, pti_hbm, dep_hbm,
                comp_hbm, ptsp_hbm, intr_hbm, pp_hbm,
                outu_hbm, outv_hbm, outd_hbm,
                cidx_v, pti_v, cam_v, dep_v, u2_v, v2_v,
                pose_rows, pt_rows, ou_v, ov_v, od_v, intr_v, pp_v,
                sem_i, sem_d, sem_g, sem_o):
        wid = lax.axis_index("s") * 2 + lax.axis_index("c")

        pltpu.sync_copy(intr_hbm, intr_v)
        pltpu.sync_copy(pp_hbm, pp_v)

        def tbase_of(h):
            return (wid + h * NW) * T

        def issue_idx(h):
            b = h % 2
            tb = tbase_of(h)
            return [
                pltpu.async_copy(cidx_hbm.at[pl.ds(tb, T)], cidx_v[b], sem_i),
                pltpu.async_copy(pti_hbm.at[pl.ds(tb, T)], pti_v[b], sem_i),
            ]

        def issue_data(h):
            b = h % 2
            tb = tbase_of(h)
            return [
                pltpu.async_copy(cam_hbm.at[pl.ds(tb, T)], cam_v[b], sem_d[b]),
                pltpu.async_copy(dep_hbm.at[pl.ds(tb, T)], dep_v[b], sem_d[b]),
                pltpu.async_copy(u2d_hbm.at[pl.ds(tb, T)], u2_v[b], sem_d[b]),
                pltpu.async_copy(v2d_hbm.at[pl.ds(tb, T)], v2_v[b], sem_d[b]),
            ]

        def issue_gathers(h):
            b = h % 2
            return [
                pltpu.async_copy(comp_hbm.at[cidx_v[b]], pose_rows[b], sem_g[b]),
                pltpu.async_copy(ptsp_hbm.at[pti_v[b]], pt_rows[b], sem_g[b]),
            ]

        def compute(h):
            b = h % 2
            tb = tbase_of(h)

            def body(i, c2):
                row = i * 16 + lax.iota(jnp.int32, 16)
                cam16 = plsc.load_gather(cam_v[b], [row])
                dep16 = plsc.load_gather(dep_v[b], [row])
                u2 = plsc.load_gather(u2_v[b], [row])
                v2 = plsc.load_gather(v2_v[b], [row])

                tx = plsc.load_gather(pose_rows[b], [row, _c16(0)])
                ty = plsc.load_gather(pose_rows[b], [row, _c16(1)])
                tz = plsc.load_gather(pose_rows[b], [row, _c16(2)])
                qx = plsc.load_gather(pose_rows[b], [row, _c16(3)])
                qy = plsc.load_gather(pose_rows[b], [row, _c16(4)])
                qz = plsc.load_gather(pose_rows[b], [row, _c16(5)])
                qw = plsc.load_gather(pose_rows[b], [row, _c16(6)])

                px = plsc.load_gather(pt_rows[b], [row, _c16(0)])
                py = plsc.load_gather(pt_rows[b], [row, _c16(1)])
                pz = plsc.load_gather(pt_rows[b], [row, _c16(2)])

                bx = 2.0 * (qy * pz - qz * py)
                by = 2.0 * (qz * px - qx * pz)
                bz = 2.0 * (qx * py - qy * px)
                cxx = px + qw * bx + (qy * bz - qz * by) + tx
                cyy = py + qw * by + (qz * bx - qx * bz) + ty
                czz = pz + qw * bz + (qx * by - qy * bx) + tz

                fx = plsc.load_gather(intr_v, [cam16, _c16(0)])
                fy = plsc.load_gather(intr_v, [cam16, _c16(1)])
                cpx = plsc.load_gather(pp_v, [cam16, _c16(0)])
                cpy = plsc.load_gather(pp_v, [cam16, _c16(1)])

                lu = cxx / czz * fx + cpx - u2
                lv = cyy / czz * fy + cpy - v2
                ld = (1.0 / (czz + EPS) - dep16) * DEPTH_W

                plsc.store_scatter(ou_v[b], [row], lu)
                plsc.store_scatter(ov_v[b], [row], lv)
                plsc.store_scatter(od_v[b], [row], ld)
                return c2

            lax.fori_loop(0, VPT, body, 0)
            return [
                pltpu.async_copy(ou_v[b], outu_hbm.at[pl.ds(tb, T)], sem_o[b]),
                pltpu.async_copy(ov_v[b], outv_hbm.at[pl.ds(tb, T)], sem_o[b]),
                pltpu.async_copy(od_v[b], outd_hbm.at[pl.ds(tb, T)], sem_o[b]),
            ]

        def wait_idx(h):
            b = h % 2
            tb = tbase_of(h)
            pltpu.make_async_copy(cidx_hbm.at[pl.ds(tb, T)], cidx_v[b], sem_i).wait()
            pltpu.make_async_copy(pti_hbm.at[pl.ds(tb, T)], pti_v[b], sem_i).wait()

        def wait_data(h):
            b = h % 2
            tb = tbase_of(h)
            pltpu.make_async_copy(cam_hbm.at[pl.ds(tb, T)], cam_v[b], sem_d[b]).wait()
            pltpu.make_async_copy(dep_hbm.at[pl.ds(tb, T)], dep_v[b], sem_d[b]).wait()
            pltpu.make_async_copy(u2d_hbm.at[pl.ds(tb, T)], u2_v[b], sem_d[b]).wait()
            pltpu.make_async_copy(v2d_hbm.at[pl.ds(tb, T)], v2_v[b], sem_d[b]).wait()

        def wait_gathers(h):
            b = h % 2
            pltpu.make_async_copy(comp_hbm.at[cidx_v[b]], pose_rows[b], sem_g[b]).wait()
            pltpu.make_async_copy(ptsp_hbm.at[pti_v[b]], pt_rows[b], sem_g[b]).wait()

        def drain_out(h):
            b = h % 2
            tb = tbase_of(h)
            pltpu.make_async_copy(ou_v[b], outu_hbm.at[pl.ds(tb, T)], sem_o[b]).wait()
            pltpu.make_async_copy(ov_v[b], outv_hbm.at[pl.ds(tb, T)], sem_o[b]).wait()
            pltpu.make_async_copy(od_v[b], outd_hbm.at[pl.ds(tb, T)], sem_o[b]).wait()

        def stage(h):
            wait_idx(h)
            issue_gathers(h)

        def compute_rest(h):
            wait_data(h)
            compute(h)

        def run(fn, h, guarded):
            if guarded:
                @pl.when(wid < rem)
                def _():
                    fn(h)
            else:
                fn(h)

        last = nslots - 1

        issue_idx(0)
        issue_data(0)
        for h in range(nslots):
            run(stage, h, h == last)
            if h >= 3:
                run(drain_out, h - 3, False)
            if h >= 1:
                run(wait_gathers, h - 1, h - 1 == last)
            if h + 1 < nslots:
                run(issue_idx, h + 1, h + 1 == last)
            if h >= 1:
                run(compute_rest, h - 1, h - 1 == last)
            if h + 1 < nslots:
                run(issue_data, h + 1, h + 1 == last)
        run(wait_gathers, last, True)
        run(compute_rest, last, True)
        for hh in (nslots - 3, nslots - 2):
            run(drain_out, hh, False)
        run(drain_out, last, True)

    return sc_call


def kernel(points_2d, camera_indices, grouping_indices, point_indices,
           camera_pps, depths_ref, intrs, points_3d, ref_poses, rel_poses):
    n = points_2d.shape[0]
    npts = points_3d.shape[0]
    u2d = points_2d[:, 0].astype(jnp.float32)
    v2d = points_2d[:, 1].astype(jnp.float32)
    cidx = (grouping_indices[:, 0] * NM
            + grouping_indices[:, 1]).astype(jnp.int32)
    cam = camera_indices.astype(jnp.int32)
    pti = point_indices.astype(jnp.int32)
    dep = depths_ref.astype(jnp.float32)
    refp = jnp.pad(ref_poses.astype(jnp.float32), ((0, 0), (0, 1)))

    p3 = points_3d.astype(jnp.float32)
    ptspf, compf = _make_prep(npts)(
        p3[:, 0], p3[:, 1], p3[:, 2], refp, rel_poses.astype(jnp.float32))
    ptsp = ptspf.reshape(npts, 8)
    comp = compf.reshape(NG * NM, 8)

    lu, lv, ld = _make_sc_call(n)(
        u2d, v2d, cidx, cam, pti, dep, comp, ptsp,
        intrs.astype(jnp.float32), camera_pps.astype(jnp.float32))
    return jnp.stack([lu, lv, ld], axis=-1)


# speedup vs baseline: 54.7368x; 1.1797x over previous
import functools

import jax
import jax.numpy as jnp
from jax import lax
from jax.experimental import pallas as pl
from jax.experimental.pallas import tpu as pltpu
from jax.experimental.pallas import tpu_sc as plsc

T = 1600
VPT = T // 16
NW = 32
TP = 2000
TC_ = 1600
NG = 20000
NM = 8
DEPTH_W = 0.1
EPS = 1e-6

_MESH = plsc.VectorSubcoreMesh(core_axis_name="c", subcore_axis_name="s",
                               num_cores=2, num_subcores=16)
_CPARAMS = pltpu.CompilerParams(use_tc_tiling_on_sc=False,
                                needs_layout_passes=False)


def _c16(v):
    return jnp.full((16,), v, jnp.int32)


def _make_prep(npts):
    npt_tiles = npts // TP
    ncmp = NG * NM
    ncmp_tiles = ncmp // TC_
    gpt = TC_ // NM

    @functools.partial(
        pl.kernel,
        out_type=[jax.ShapeDtypeStruct((npts * 8,), jnp.float32),
                  jax.ShapeDtypeStruct((ncmp * 8,), jnp.float32)],
        mesh=_MESH,
        compiler_params=_CPARAMS,
        scratch_types=[
            pltpu.VMEM((TP,), jnp.float32),
            pltpu.VMEM((TP,), jnp.float32),
            pltpu.VMEM((TP,), jnp.float32),
            pltpu.VMEM((8 * TP,), jnp.float32),
            pltpu.VMEM((gpt, 8), jnp.float32),
            pltpu.VMEM((8, 7), jnp.float32),
            pltpu.VMEM((8 * TC_,), jnp.float32),
            pltpu.SemaphoreType.DMA,
        ],
    )
    def prep_call(x_hbm, y_hbm, z_hbm, refp_hbm, rel_hbm, out_hbm, comp_hbm,
                  x_v, y_v, z_v, row_v, ref_v, rel_v, comp_v, sem):
        wid = lax.axis_index("s") * 2 + lax.axis_index("c")
        pltpu.sync_copy(rel_hbm, rel_v)

        n_tiles_w = (npt_tiles + NW - 1 - wid) // NW

        def tile_body(k, carry):
            tbase = (wid + k * NW) * TP
            cps = [
                pltpu.async_copy(x_hbm.at[pl.ds(tbase, TP)], x_v, sem),
                pltpu.async_copy(y_hbm.at[pl.ds(tbase, TP)], y_v, sem),
                pltpu.async_copy(z_hbm.at[pl.ds(tbase, TP)], z_v, sem),
            ]
            for cp in cps:
                cp.wait()

            def vec_body(i, c2):
                row = i * 16 + lax.iota(jnp.int32, 16)
                plsc.store_scatter(row_v, [8 * row], plsc.load_gather(x_v, [row]))
                plsc.store_scatter(row_v, [8 * row + 1], plsc.load_gather(y_v, [row]))
                plsc.store_scatter(row_v, [8 * row + 2], plsc.load_gather(z_v, [row]))
                return c2

            lax.fori_loop(0, TP // 16, vec_body, 0)
            pltpu.sync_copy(row_v, out_hbm.at[pl.ds(8 * tbase, 8 * TP)])
            return carry

        lax.fori_loop(0, n_tiles_w, tile_body, 0)

        n_ctiles_w = (ncmp_tiles + NW - 1 - wid) // NW

        def ctile_body(k, carry):
            t = wid + k * NW
            pltpu.sync_copy(refp_hbm.at[pl.ds(gpt * t, gpt), :], ref_v)

            def cvec_body(i, c2):
                lrow = i * 16 + lax.iota(jnp.int32, 16)
                gl = lax.shift_right_logical(lrow, 3)
                m = lax.bitwise_and(lrow, 7)

                ttx = plsc.load_gather(ref_v, [gl, _c16(0)])
                tty = plsc.load_gather(ref_v, [gl, _c16(1)])
                ttz = plsc.load_gather(ref_v, [gl, _c16(2)])
                tqx = plsc.load_gather(ref_v, [gl, _c16(3)])
                tqy = plsc.load_gather(ref_v, [gl, _c16(4)])
                tqz = plsc.load_gather(ref_v, [gl, _c16(5)])
                tqw = plsc.load_gather(ref_v, [gl, _c16(6)])

                rtx = plsc.load_gather(rel_v, [m, _c16(0)])
                rty = plsc.load_gather(rel_v, [m, _c16(1)])
                rtz = plsc.load_gather(rel_v, [m, _c16(2)])
                rqx = plsc.load_gather(rel_v, [m, _c16(3)])
                rqy = plsc.load_gather(rel_v, [m, _c16(4)])
                rqz = plsc.load_gather(rel_v, [m, _c16(5)])
                rqw = plsc.load_gather(rel_v, [m, _c16(6)])

                qw = rqw * tqw - rqx * tqx - rqy * tqy - rqz * tqz
                qx = rqw * tqx + rqx * tqw + rqy * tqz - rqz * tqy
                qy = rqw * tqy + rqy * tqw + rqz * tqx - rqx * tqz
                qz = rqw * tqz + rqz * tqw + rqx * tqy - rqy * tqx

                ax = 2.0 * (rqy * ttz - rqz * tty)
                ay = 2.0 * (rqz * ttx - rqx * ttz)
                az = 2.0 * (rqx * tty - rqy * ttx)
                tx = ttx + rqw * ax + (rqy * az - rqz * ay) + rtx
                ty = tty + rqw * ay + (rqz * ax - rqx * az) + rty
                tz = ttz + rqw * az + (rqx * ay - rqy * ax) + rtz

                plsc.store_scatter(comp_v, [8 * lrow], tx)
                plsc.store_scatter(comp_v, [8 * lrow + 1], ty)
                plsc.store_scatter(comp_v, [8 * lrow + 2], tz)
                plsc.store_scatter(comp_v, [8 * lrow + 3], qx)
                plsc.store_scatter(comp_v, [8 * lrow + 4], qy)
                plsc.store_scatter(comp_v, [8 * lrow + 5], qz)
                plsc.store_scatter(comp_v, [8 * lrow + 6], qw)
                return c2

            lax.fori_loop(0, TC_ // 16, cvec_body, 0)
            pltpu.sync_copy(comp_v, comp_hbm.at[pl.ds(8 * TC_ * t, 8 * TC_)])
            return carry

        lax.fori_loop(0, n_ctiles_w, ctile_body, 0)

    return prep_call


def _make_sc_call(n):
    nt = n // T
    base_slots = nt // NW
    rem = nt - base_slots * NW
    nslots = base_slots + 1

    @functools.partial(
        pl.kernel,
        out_type=jax.ShapeDtypeStruct((4 * n,), jnp.float32),
        mesh=_MESH,
        compiler_params=_CPARAMS,
        scratch_types=[
            [pltpu.VMEM((T,), jnp.int32)] * 2,
            [pltpu.VMEM((T,), jnp.int32)] * 2,
            [pltpu.VMEM((T,), jnp.int32)] * 2,
            [pltpu.VMEM((T,), jnp.float32)] * 2,
            [pltpu.VMEM((T,), jnp.float32)] * 2,
            [pltpu.VMEM((T,), jnp.float32)] * 2,
            [pltpu.VMEM((T, 8), jnp.float32)] * 2,
            [pltpu.VMEM((T, 8), jnp.float32)] * 2,
            [pltpu.VMEM((T,), jnp.float32)] * 2,
            [pltpu.VMEM((T,), jnp.float32)] * 2,
            [pltpu.VMEM((T,), jnp.float32)] * 2,
            pltpu.VMEM((8, 2), jnp.float32),
            pltpu.VMEM((8, 2), jnp.float32),
            pltpu.SemaphoreType.DMA,
            [pltpu.SemaphoreType.DMA] * 2,
            [pltpu.SemaphoreType.DMA] * 2,
            [pltpu.SemaphoreType.DMA] * 2,
        ],
    )
    def sc_call(u2d_hbm, v2d_hbm, cidx_hbm, cam_hbm, pti_hbm, dep_hbm,
                comp_hbm, ptsp_hbm, intr_hbm, pp_hbm, out_hbm,
                cidx_v, pti_v, cam_v, dep_v, u2_v, v2_v,
                pose_rows, pt_rows, ou_v, ov_v, od_v, intr_v, pp_v,
                sem_i, sem_d, sem_g, sem_o):
        wid = lax.axis_index("s") * 2 + lax.axis_index("c")

        pltpu.sync_copy(intr_hbm, intr_v)
        pltpu.sync_copy(pp_hbm, pp_v)

        def tbase_of(h):
            return (wid + h * NW) * T

        def issue_idx(h):
            b = h % 2
            tb = tbase_of(h)
            return [
                pltpu.async_copy(cidx_hbm.at[pl.ds(tb, T)], cidx_v[b], sem_i),
                pltpu.async_copy(pti_hbm.at[pl.ds(tb, T)], pti_v[b], sem_i),
            ]

        def issue_data(h):
            b = h % 2
            tb = tbase_of(h)
            return [
                pltpu.async_copy(cam_hbm.at[pl.ds(tb, T)], cam_v[b], sem_d[b]),
                pltpu.async_copy(dep_hbm.at[pl.ds(tb, T)], dep_v[b], sem_d[b]),
                pltpu.async_copy(u2d_hbm.at[pl.ds(tb, T)], u2_v[b], sem_d[b]),
                pltpu.async_copy(v2d_hbm.at[pl.ds(tb, T)], v2_v[b], sem_d[b]),
            ]

        def issue_gathers(h):
            b = h % 2
            return [
                pltpu.async_copy(comp_hbm.at[cidx_v[b]], pose_rows[b], sem_g[b]),
                pltpu.async_copy(ptsp_hbm.at[pti_v[b]], pt_rows[b], sem_g[b]),
            ]

        def compute(h):
            b = h % 2
            tb = tbase_of(h)

            def body(i, c2):
                row = i * 16 + lax.iota(jnp.int32, 16)
                cam16 = plsc.load_gather(cam_v[b], [row])
                dep16 = plsc.load_gather(dep_v[b], [row])
                u2 = plsc.load_gather(u2_v[b], [row])
                v2 = plsc.load_gather(v2_v[b], [row])

                tx = plsc.load_gather(pose_rows[b], [row, _c16(0)])
                ty = plsc.load_gather(pose_rows[b], [row, _c16(1)])
                tz = plsc.load_gather(pose_rows[b], [row, _c16(2)])
                qx = plsc.load_gather(pose_rows[b], [row, _c16(3)])
                qy = plsc.load_gather(pose_rows[b], [row, _c16(4)])
                qz = plsc.load_gather(pose_rows[b], [row, _c16(5)])
                qw = plsc.load_gather(pose_rows[b], [row, _c16(6)])

                px = plsc.load_gather(pt_rows[b], [row, _c16(0)])
                py = plsc.load_gather(pt_rows[b], [row, _c16(1)])
                pz = plsc.load_gather(pt_rows[b], [row, _c16(2)])

                bx = 2.0 * (qy * pz - qz * py)
                by = 2.0 * (qz * px - qx * pz)
                bz = 2.0 * (qx * py - qy * px)
                cxx = px + qw * bx + (qy * bz - qz * by) + tx
                cyy = py + qw * by + (qz * bx - qx * bz) + ty
                czz = pz + qw * bz + (qx * by - qy * bx) + tz

                fx = plsc.load_gather(intr_v, [cam16, _c16(0)])
                fy = plsc.load_gather(intr_v, [cam16, _c16(1)])
                cpx = plsc.load_gather(pp_v, [cam16, _c16(0)])
                cpy = plsc.load_gather(pp_v, [cam16, _c16(1)])

                lu = cxx / czz * fx + cpx - u2
                lv = cyy / czz * fy + cpy - v2
                ld = (1.0 / (czz + EPS) - dep16) * DEPTH_W

                plsc.store_scatter(ou_v[b], [row], lu)
                plsc.store_scatter(ov_v[b], [row], lv)
                plsc.store_scatter(od_v[b], [row], ld)
                return c2

            lax.fori_loop(0, VPT, body, 0)
            return [
                pltpu.async_copy(ou_v[b], out_hbm.at[pl.ds(tb, T)], sem_o[b]),
                pltpu.async_copy(ov_v[b], out_hbm.at[pl.ds(n + tb, T)], sem_o[b]),
                pltpu.async_copy(od_v[b], out_hbm.at[pl.ds(2 * n + tb, T)], sem_o[b]),
            ]

        def wait_idx(h):
            b = h % 2
            tb = tbase_of(h)
            pltpu.make_async_copy(cidx_hbm.at[pl.ds(tb, T)], cidx_v[b], sem_i).wait()
            pltpu.make_async_copy(pti_hbm.at[pl.ds(tb, T)], pti_v[b], sem_i).wait()

        def wait_data(h):
            b = h % 2
            tb = tbase_of(h)
            pltpu.make_async_copy(cam_hbm.at[pl.ds(tb, T)], cam_v[b], sem_d[b]).wait()
            pltpu.make_async_copy(dep_hbm.at[pl.ds(tb, T)], dep_v[b], sem_d[b]).wait()
            pltpu.make_async_copy(u2d_hbm.at[pl.ds(tb, T)], u2_v[b], sem_d[b]).wait()
            pltpu.make_async_copy(v2d_hbm.at[pl.ds(tb, T)], v2_v[b], sem_d[b]).wait()

        def wait_gathers(h):
            b = h % 2
            pltpu.make_async_copy(comp_hbm.at[cidx_v[b]], pose_rows[b], sem_g[b]).wait()
            pltpu.make_async_copy(ptsp_hbm.at[pti_v[b]], pt_rows[b], sem_g[b]).wait()

        def drain_out(h):
            b = h % 2
            tb = tbase_of(h)
            pltpu.make_async_copy(ou_v[b], out_hbm.at[pl.ds(tb, T)], sem_o[b]).wait()
            pltpu.make_async_copy(ov_v[b], out_hbm.at[pl.ds(n + tb, T)], sem_o[b]).wait()
            pltpu.make_async_copy(od_v[b], out_hbm.at[pl.ds(2 * n + tb, T)], sem_o[b]).wait()

        def stage(h):
            wait_idx(h)
            issue_gathers(h)

        def compute_rest(h):
            wait_data(h)
            compute(h)

        def run(fn, h, guarded):
            if guarded:
                @pl.when(wid < rem)
                def _():
                    fn(h)
            else:
                fn(h)

        last = nslots - 1

        issue_idx(0)
        issue_data(0)
        for h in range(nslots):
            run(stage, h, h == last)
            if h >= 3:
                run(drain_out, h - 3, False)
            if h >= 1:
                run(wait_gathers, h - 1, h - 1 == last)
            if h + 1 < nslots:
                run(issue_idx, h + 1, h + 1 == last)
            if h >= 1:
                run(compute_rest, h - 1, h - 1 == last)
            if h + 1 < nslots:
                run(issue_data, h + 1, h + 1 == last)
        run(wait_gathers, last, True)
        run(compute_rest, last, True)
        for hh in (nslots - 3, nslots - 2):
            run(drain_out, hh, False)
        run(drain_out, last, True)

    return sc_call


def kernel(points_2d, camera_indices, grouping_indices, point_indices,
           camera_pps, depths_ref, intrs, points_3d, ref_poses, rel_poses):
    n = points_2d.shape[0]
    npts = points_3d.shape[0]
    u2d = points_2d[:, 0].astype(jnp.float32)
    v2d = points_2d[:, 1].astype(jnp.float32)
    cidx = (grouping_indices[:, 0] * NM
            + grouping_indices[:, 1]).astype(jnp.int32)
    cam = camera_indices.astype(jnp.int32)
    pti = point_indices.astype(jnp.int32)
    dep = depths_ref.astype(jnp.float32)
    refp = jnp.pad(ref_poses.astype(jnp.float32), ((0, 0), (0, 1)))

    p3 = points_3d.astype(jnp.float32)
    ptspf, compf = _make_prep(npts)(
        p3[:, 0], p3[:, 1], p3[:, 2], refp, rel_poses.astype(jnp.float32))
    ptsp = ptspf.reshape(npts, 8)
    comp = compf.reshape(NG * NM, 8)

    out4 = _make_sc_call(n)(
        u2d, v2d, cidx, cam, pti, dep, comp, ptsp,
        intrs.astype(jnp.float32), camera_pps.astype(jnp.float32))
    return out4.reshape(4, n).T[:, :3]
